# baseline (speedup 1.0000x reference)
import jax
import jax.numpy as jnp
from jax import lax
from jax.experimental import pallas as pl
from jax.experimental.pallas import tpu as pltpu
from jax.experimental.pallas import tpu_sc as plsc

_N = 10000
_E = 320000
_NPAD = 10240
_NSUB = 16
_EC = _E // _NSUB
_ROWS = 160
_ECP = _ROWS * 128
_FULL = _EC // 128
_REM = _EC - _FULL * 128
_NS = _NPAD // _NSUB
_D = 128
_H1 = 256
_H2 = 128
_EMB = 64
_XCH = 160

_T1_RD = 0
_T1_WX = 2048
_T1_W = 2304
_T1_P = 4352
_T2_RD = 0
_T2_T1 = 4096
_T2_SV = 4608
_T2_SVS = 4864
_T2_B1 = 4880
_T2_W = 5136
_T2_P = 7184
_O_RD = 0
_O_T2 = 2048
_O_B2 = 2176
_O_W = 2304
_O_P = 2816
_F_RD = 0
_F_SUM = 1024
_F_B = 1088
_F_OUT = 1152


def _rsqrt16(d):
    i = lax.bitcast_convert_type(d, jnp.int32)
    i = jnp.int32(0x5F3759DF) - lax.shift_right_arithmetic(i, 1)
    y = lax.bitcast_convert_type(i, jnp.float32)
    for _ in range(3):
        y = y * (1.5 - 0.5 * d * y * y)
    return y


def _scatter_pass(vals, idx2d, acc, sem, per):
    def body(g, _):
        j0 = g * per
        descs = [
            pltpu.async_copy(vals.at[pl.ds((j0 + t) * 128, 128)],
                             acc.at[idx2d.at[j0 + t]], sem, add=True)
            for t in range(per)
        ]
        for d in descs:
            d.wait()
        return 0
    lax.fori_loop(0, _ROWS // per, body, 0)


def _sc_body(src_hbm, dst_hbm, x_hbm, w1_hbm, b1_hbm, w2_hbm, b2_hbm,
             wo_hbm, bo_hbm, out_hbm,
             src_flat, dst_flat, src2d, dst2d, vals, cfull,
             zbuf, abuf, cbuf, sbuf, tbuf, sem, sem2, sem3,
             acc_a, acc_b, carr, vcarr,
             wxstage, svstage, t1stage, t2stage, outstage):
    wid = lax.axis_index("s")
    ebase = wid * _EC
    nbase = wid * _NS
    nsl = pl.ds(nbase, _NS)

    def z16(i, _):
        zbuf[pl.ds(i * 16, 16)] = jnp.zeros((16,), jnp.float32)
        return 0
    lax.fori_loop(0, _NS // 16, z16, 0)
    pltpu.sync_copy(zbuf, acc_a.at[nsl])
    pltpu.sync_copy(zbuf, acc_b.at[nsl])

    d_src = pltpu.async_copy(src_hbm.at[pl.ds(ebase, _EC)],
                             src_flat.at[pl.ds(0, _EC)], sem)
    d_dst = pltpu.async_copy(dst_hbm.at[pl.ds(ebase, _EC)],
                             dst_flat.at[pl.ds(0, _EC)], sem)
    d_src.wait()
    d_dst.wait()

    def repack(j, _):
        def inner(k, _):
            s = pl.ds(j * 128 + k * 16, 16)
            d = pl.ds(k * 16, 16)
            src2d[j, d] = src_flat[s]
            dst2d[j, d] = dst_flat[s]
            return 0
        lax.fori_loop(0, 8, inner, 0)
        return 0
    lax.fori_loop(0, _FULL, repack, 0)
    pv = jnp.int32(_N) + (wid * 16 + lax.iota(jnp.int32, 16)) % (_NPAD - _N)
    for k in range(_REM // 16):
        s = pl.ds(_FULL * 128 + k * 16, 16)
        src2d[_FULL, pl.ds(k * 16, 16)] = src_flat[s]
        dst2d[_FULL, pl.ds(k * 16, 16)] = dst_flat[s]
    for j in range(_FULL, _ROWS):
        for k in range((_REM // 16) if j == _FULL else 0, 8):
            src2d[j, pl.ds(k * 16, 16)] = pv
            dst2d[j, pl.ds(k * 16, 16)] = pv

    def ones16(i, _):
        vals[pl.ds(i * 16, 16)] = jnp.ones((16,), jnp.float32)
        return 0
    lax.fori_loop(0, _EC // 16, ones16, 0)
    def zeros16(i, _):
        vals[pl.ds(i * 16, 16)] = jnp.zeros((16,), jnp.float32)
        return 0
    lax.fori_loop(_EC // 16, _ECP // 16, zeros16, 0)

    plsc.subcore_barrier()

    def hist(g, _):
        j0 = g * 4
        descs = []
        for t in range(4):
            vsl = vals.at[pl.ds((j0 + t) * 128, 128)]
            descs.append(pltpu.async_copy(vsl, acc_a.at[src2d.at[j0 + t]],
                                          sem, add=True))
            descs.append(pltpu.async_copy(vsl, acc_b.at[dst2d.at[j0 + t]],
                                          sem, add=True))
        for d in descs:
            d.wait()
        return 0
    lax.fori_loop(0, _ROWS // 4, hist, 0)
    plsc.subcore_barrier()

    pltpu.sync_copy(acc_a.at[nsl], sbuf)
    def fin_a(i, _):
        s = pl.ds(i * 16, 16)
        abuf[s] = _rsqrt16(jnp.maximum(sbuf[s], 1.0))
        return 0
    lax.fori_loop(0, _NS // 16, fin_a, 0)
    pltpu.sync_copy(acc_b.at[nsl], sbuf)
    def fin_c(i, _):
        s = pl.ds(i * 16, 16)
        cbuf[s] = _rsqrt16(jnp.maximum(sbuf[s], 1.0))
        return 0
    lax.fori_loop(0, _NS // 16, fin_c, 0)
    pltpu.sync_copy(cbuf, carr.at[nsl])
    pltpu.sync_copy(zbuf, acc_a.at[nsl])
    pltpu.sync_copy(zbuf, acc_b.at[nsl])
    plsc.subcore_barrier()

    pltpu.sync_copy(carr, cfull)
    def gat(i, _):
        s = pl.ds(i * 16, 16)
        vals[s] = plsc.load_gather(cfull, [dst_flat[s]])
        return 0
    lax.fori_loop(0, _EC // 16, gat, 0)
    _scatter_pass(vals, src2d, acc_a, sem, 8)
    plsc.subcore_barrier()

    pltpu.sync_copy(acc_a.at[nsl], sbuf)
    def fin_v(i, psum):
        s = pl.ds(i * 16, 16)
        vv = abuf[s] * sbuf[s]
        tbuf[s] = vv
        cbuf[s] = vv * cbuf[s]
        return psum + vv
    psum = lax.fori_loop(0, _NS // 16, fin_v,
                         jnp.zeros((16,), jnp.float32))
    zbuf[pl.ds(0, 16)] = psum
    pltpu.sync_copy(zbuf.at[pl.ds(0, 16)], svstage.at[pl.ds(wid * 16, 16)])
    pltpu.sync_copy(cbuf, vcarr.at[nsl])
    plsc.subcore_barrier()

    pltpu.sync_copy(vcarr, cfull)
    lax.fori_loop(0, _EC // 16, gat, 0)
    _scatter_pass(vals, src2d, acc_b, sem, 8)
    plsc.subcore_barrier()

    pltpu.sync_copy(acc_b.at[nsl], sbuf)
    def fin_w(i, _):
        s = pl.ds(i * 16, 16)
        tbuf[s] = abuf[s] * sbuf[s]
        return 0
    lax.fori_loop(0, _NS // 16, fin_w, 0)

    for t in range(8):
        zbuf[pl.ds(16 + t * 16, 16)] = jnp.zeros((16,), jnp.float32)
    _XB = 80 * _D
    sems = (sem2, sem3)

    def xsrc(ch):
        start = jnp.minimum(nbase + ch * 80, _N - 80) * _D
        return x_hbm.at[pl.ds(start, _XB)]

    def xdst(ch):
        return vals.at[pl.ds((ch % 2) * _XB, _XB)]

    descs = [None] * 9
    descs[0] = pltpu.async_copy(xsrc(0), xdst(0), sems[0])
    for ch in range(8):
        if ch + 1 < 8:
            descs[ch + 1] = pltpu.async_copy(xsrc(ch + 1), xdst(ch + 1),
                                             sems[(ch + 1) % 2])
        descs[ch].wait()
        half = (ch % 2) * _XB

        def rowbody(r16, acc, ch=ch, half=half):
            wvec = tbuf[pl.ds(ch * 80 + r16 * 16, 16)]
            for rr in range(16):
                ws = wvec[rr]
                base = half + (r16 * 16 + rr) * _D
                acc = tuple(acc[t] + ws * vals[pl.ds(base + t * 16, 16)]
                            for t in range(8))
            return acc
        acc = lax.fori_loop(0, 5, rowbody,
                            tuple(jnp.zeros((16,), jnp.float32)
                                  for _ in range(8)))
        for t in range(8):
            s = pl.ds(16 + t * 16, 16)
            zbuf[s] = zbuf[s] + acc[t]

    pltpu.sync_copy(zbuf.at[pl.ds(16, _D)],
                    wxstage.at[pl.ds(wid * _D, _D)])
    plsc.subcore_barrier()

    def sum_rows(src_off, nrows, rowlen, dst_off):
        for jj in range(rowlen // 16):
            def body(r, acc):
                return acc + cfull[pl.ds(src_off + r * rowlen + jj * 16, 16)]
            s = lax.fori_loop(0, nrows, body, jnp.zeros((16,), jnp.float32))
            cfull[pl.ds(dst_off + jj * 16, 16)] = s

    def mm_partial(scal_off, w_off, kcount, outlen, out_off):
        kvec = cfull[pl.ds(scal_off, 16)]
        for jj in range(outlen // 16):
            s = jnp.zeros((16,), jnp.float32)
            for k in range(kcount):
                s = s + kvec[k] * cfull[pl.ds(w_off + k * outlen + jj * 16, 16)]
            cfull[pl.ds(out_off + jj * 16, 16)] = s

    pltpu.sync_copy(wxstage, cfull.at[pl.ds(_T1_RD, _NSUB * _D)])
    sum_rows(_T1_RD, _NSUB, _D, _T1_WX)
    pltpu.sync_copy(w1_hbm.at[pl.ds(wid * 8 * _H1, 8 * _H1)],
                    cfull.at[pl.ds(_T1_W, 8 * _H1)])
    mm_partial(_T1_WX + wid * 8, _T1_W, 8, _H1, _T1_P)
    pltpu.sync_copy(cfull.at[pl.ds(_T1_P, _H1)],
                    t1stage.at[pl.ds(wid * _H1, _H1)])
    plsc.subcore_barrier()

    pltpu.sync_copy(t1stage, cfull.at[pl.ds(_T2_RD, _NSUB * _H1)])
    sum_rows(_T2_RD, _NSUB, _H1, _T2_T1)
    pltpu.sync_copy(svstage, cfull.at[pl.ds(_T2_SV, _NSUB * 16)])
    sum_rows(_T2_SV, _NSUB, 16, _T2_SVS)
    sv = jnp.sum(cfull[pl.ds(_T2_SVS, 16)])
    pltpu.sync_copy(b1_hbm, cfull.at[pl.ds(_T2_B1, _H1)])
    for jj in range(_H1 // 16):
        s = pl.ds(_T2_T1 + jj * 16, 16)
        cfull[s] = cfull[s] + sv * cfull[pl.ds(_T2_B1 + jj * 16, 16)]
    pltpu.sync_copy(w2_hbm.at[pl.ds(wid * 16 * _H2, 16 * _H2)],
                    cfull.at[pl.ds(_T2_W, 16 * _H2)])
    mm_partial(_T2_T1 + wid * 16, _T2_W, 16, _H2, _T2_P)
    pltpu.sync_copy(cfull.at[pl.ds(_T2_P, _H2)],
                    t2stage.at[pl.ds(wid * _H2, _H2)])
    plsc.subcore_barrier()

    pltpu.sync_copy(t2stage, cfull.at[pl.ds(_O_RD, _NSUB * _H2)])
    sum_rows(_O_RD, _NSUB, _H2, _O_T2)
    pltpu.sync_copy(b2_hbm, cfull.at[pl.ds(_O_B2, _H2)])
    for jj in range(_H2 // 16):
        s = pl.ds(_O_T2 + jj * 16, 16)
        cfull[s] = cfull[s] + jnp.float32(_N) * cfull[pl.ds(_O_B2 + jj * 16, 16)]
    pltpu.sync_copy(wo_hbm.at[pl.ds(wid * 8 * _EMB, 8 * _EMB)],
                    cfull.at[pl.ds(_O_W, 8 * _EMB)])
    mm_partial(_O_T2 + wid * 8, _O_W, 8, _EMB, _O_P)
    pltpu.sync_copy(cfull.at[pl.ds(_O_P, _EMB)],
                    outstage.at[pl.ds(wid * _EMB, _EMB)])
    plsc.subcore_barrier()

    @pl.when(wid == 0)
    def _():
        pltpu.sync_copy(outstage, cfull.at[pl.ds(_F_RD, _NSUB * _EMB)])
        sum_rows(_F_RD, _NSUB, _EMB, _F_SUM)
        pltpu.sync_copy(bo_hbm, cfull.at[pl.ds(_F_B, _EMB)])
        for jj in range(_EMB // 16):
            cfull[pl.ds(_F_OUT + jj * 16, 16)] = (
                cfull[pl.ds(_F_SUM + jj * 16, 16)]
                + jnp.float32(_N) * cfull[pl.ds(_F_B + jj * 16, 16)])
        pltpu.sync_copy(cfull.at[pl.ds(_F_OUT, _EMB)], out_hbm)


_SCRATCH = [
        pltpu.VMEM((_ECP,), jnp.int32),
        pltpu.VMEM((_ECP,), jnp.int32),
        pltpu.VMEM((_ROWS, 128), jnp.int32),
        pltpu.VMEM((_ROWS, 128), jnp.int32),
        pltpu.VMEM((_ECP,), jnp.float32),
        pltpu.VMEM((_NPAD,), jnp.float32),
        pltpu.VMEM((_NS,), jnp.float32),
        pltpu.VMEM((_NS,), jnp.float32),
        pltpu.VMEM((_NS,), jnp.float32),
        pltpu.VMEM((_NS,), jnp.float32),
        pltpu.VMEM((_NS,), jnp.float32),
        pltpu.SemaphoreType.DMA,
        pltpu.SemaphoreType.DMA,
        pltpu.SemaphoreType.DMA,
        pltpu.VMEM_SHARED((_NPAD,), jnp.float32),
        pltpu.VMEM_SHARED((_NPAD,), jnp.float32),
        pltpu.VMEM_SHARED((_NPAD,), jnp.float32),
        pltpu.VMEM_SHARED((_NPAD,), jnp.float32),
        pltpu.VMEM_SHARED((_NSUB * _D,), jnp.float32),
        pltpu.VMEM_SHARED((_NSUB * 16,), jnp.float32),
        pltpu.VMEM_SHARED((_NSUB * _H1,), jnp.float32),
        pltpu.VMEM_SHARED((_NSUB * _H2,), jnp.float32),
        pltpu.VMEM_SHARED((_NSUB * _EMB,), jnp.float32),
]

_sc_fn = pl.kernel(
    _sc_body,
    out_type=jax.ShapeDtypeStruct((_EMB,), jnp.float32),
    mesh=plsc.VectorSubcoreMesh(core_axis_name="c", subcore_axis_name="s",
                                num_cores=1, num_subcores=_NSUB),
    compiler_params=pltpu.CompilerParams(needs_layout_passes=False),
    scratch_types=_SCRATCH,
)


def kernel(x, edge_index, W1, b1, W2, b2, Wout, bout):
    return _sc_fn(edge_index[0], edge_index[1], x.reshape(-1),
                  W1.reshape(-1), b1, W2.reshape(-1), b2,
                  Wout.reshape(-1), bout)


# speedup vs baseline: 1.1814x; 1.1814x over previous
import jax
import jax.numpy as jnp
from jax import lax
from jax.experimental import pallas as pl
from jax.experimental.pallas import tpu as pltpu
from jax.experimental.pallas import tpu_sc as plsc

_N = 10000
_E = 320000
_NPAD = 10240
_NSUB = 16
_EC = _E // _NSUB
_ROWS = 160
_ECP = _ROWS * 128
_FULL = _EC // 128
_REM = _EC - _FULL * 128
_NS = _NPAD // _NSUB


def _rsqrt16(d):
    i = lax.bitcast_convert_type(d, jnp.int32)
    i = jnp.int32(0x5F3759DF) - lax.shift_right_arithmetic(i, 1)
    y = lax.bitcast_convert_type(i, jnp.float32)
    for _ in range(3):
        y = y * (1.5 - 0.5 * d * y * y)
    return y


def _drain_row(vals, idx2d, acc, sem, j):
    pltpu.make_async_copy(vals.at[pl.ds(j * 128, 128)],
                          acc.at[idx2d.at[j]], sem).wait()


def _gs_pass(vals, dst_flat, src2d, cfull, acc, sem):
    G = 8
    def body(g, _):
        j0 = g * G
        def grow(j, _):
            def gchunk(k, _):
                s = pl.ds(j * 128 + k * 16, 16)
                vals[s] = plsc.load_gather(cfull, [dst_flat[s]])
                return 0
            lax.fori_loop(0, 8, gchunk, 0)
            return 0
        lax.fori_loop(j0, j0 + G, grow, 0)
        for t in range(G):
            pltpu.async_copy(vals.at[pl.ds((j0 + t) * 128, 128)],
                             acc.at[src2d.at[j0 + t]], sem, add=True)
        @pl.when(g > 0)
        def _():
            for t in range(G):
                _drain_row(vals, src2d, acc, sem, j0 - G + t)
        return 0
    lax.fori_loop(0, _ROWS // G, body, 0)
    for t in range(G):
        _drain_row(vals, src2d, acc, sem, _ROWS - G + t)


def _sc_body(src_hbm, dst_hbm, v_hbm, w_hbm,
             src_flat, dst_flat, src2d, dst2d, vals, cfull,
             zbuf, abuf, cbuf, sbuf, tbuf, sem,
             acc_a, acc_b, carr, vcarr):
    wid = lax.axis_index("s")
    ebase = wid * _EC
    nbase = wid * _NS
    nsl = pl.ds(nbase, _NS)

    def z16(i, _):
        zbuf[pl.ds(i * 16, 16)] = jnp.zeros((16,), jnp.float32)
        return 0
    lax.fori_loop(0, _NS // 16, z16, 0)
    pltpu.sync_copy(zbuf, acc_a.at[nsl])
    pltpu.sync_copy(zbuf, acc_b.at[nsl])

    d_src = pltpu.async_copy(src_hbm.at[pl.ds(ebase, _EC)],
                             src_flat.at[pl.ds(0, _EC)], sem)
    d_dst = pltpu.async_copy(dst_hbm.at[pl.ds(ebase, _EC)],
                             dst_flat.at[pl.ds(0, _EC)], sem)
    d_src.wait()
    d_dst.wait()

    def repack(j, _):
        def inner(k, _):
            s = pl.ds(j * 128 + k * 16, 16)
            d = pl.ds(k * 16, 16)
            src2d[j, d] = src_flat[s]
            dst2d[j, d] = dst_flat[s]
            return 0
        lax.fori_loop(0, 8, inner, 0)
        return 0
    lax.fori_loop(0, _FULL, repack, 0)
    pv = jnp.int32(_N) + (wid * 16 + lax.iota(jnp.int32, 16)) % (_NPAD - _N)
    for k in range(_REM // 16):
        s = pl.ds(_FULL * 128 + k * 16, 16)
        src2d[_FULL, pl.ds(k * 16, 16)] = src_flat[s]
        dst2d[_FULL, pl.ds(k * 16, 16)] = dst_flat[s]
    for j in range(_FULL, _ROWS):
        for k in range((_REM // 16) if j == _FULL else 0, 8):
            src2d[j, pl.ds(k * 16, 16)] = pv
            dst2d[j, pl.ds(k * 16, 16)] = pv
    def padfill(i, _):
        dst_flat[pl.ds(i * 16, 16)] = jnp.full((16,), _NPAD, jnp.int32)
        return 0
    lax.fori_loop(_EC // 16, _ECP // 16, padfill, 0)

    def ones16(i, _):
        vals[pl.ds(i * 16, 16)] = jnp.ones((16,), jnp.float32)
        return 0
    lax.fori_loop(0, _EC // 16, ones16, 0)
    def zeros16(i, _):
        vals[pl.ds(i * 16, 16)] = jnp.zeros((16,), jnp.float32)
        return 0
    lax.fori_loop(_EC // 16, _ECP // 16, zeros16, 0)

    plsc.subcore_barrier()

    def hist(g, _):
        j0 = g * 4
        for t in range(4):
            vsl = vals.at[pl.ds((j0 + t) * 128, 128)]
            pltpu.async_copy(vsl, acc_a.at[src2d.at[j0 + t]], sem, add=True)
            pltpu.async_copy(vsl, acc_b.at[dst2d.at[j0 + t]], sem, add=True)
        @pl.when(g > 0)
        def _():
            for t in range(4):
                _drain_row(vals, src2d, acc_a, sem, j0 - 4 + t)
                _drain_row(vals, dst2d, acc_b, sem, j0 - 4 + t)
        return 0
    lax.fori_loop(0, _ROWS // 4, hist, 0)
    for t in range(4):
        _drain_row(vals, src2d, acc_a, sem, _ROWS - 4 + t)
        _drain_row(vals, dst2d, acc_b, sem, _ROWS - 4 + t)
    plsc.subcore_barrier()

    pltpu.sync_copy(acc_a.at[nsl], sbuf)
    def fin_a(i, _):
        s = pl.ds(i * 16, 16)
        abuf[s] = _rsqrt16(jnp.maximum(sbuf[s], 1.0))
        return 0
    lax.fori_loop(0, _NS // 16, fin_a, 0)
    pltpu.sync_copy(acc_b.at[nsl], sbuf)
    def fin_c(i, _):
        s = pl.ds(i * 16, 16)
        cbuf[s] = _rsqrt16(jnp.maximum(sbuf[s], 1.0))
        return 0
    lax.fori_loop(0, _NS // 16, fin_c, 0)
    pltpu.sync_copy(cbuf, carr.at[nsl])
    pltpu.sync_copy(zbuf, acc_a.at[nsl])
    pltpu.sync_copy(zbuf, acc_b.at[nsl])
    plsc.subcore_barrier()

    pltpu.sync_copy(carr, cfull.at[pl.ds(0, _NPAD)])
    cfull[pl.ds(_NPAD, 16)] = jnp.zeros((16,), jnp.float32)
    _gs_pass(vals, dst_flat, src2d, cfull, acc_a, sem)
    plsc.subcore_barrier()

    pltpu.sync_copy(acc_a.at[nsl], sbuf)
    def fin_v(i, _):
        s = pl.ds(i * 16, 16)
        vv = abuf[s] * sbuf[s]
        tbuf[s] = vv
        cbuf[s] = vv * cbuf[s]
        return 0
    lax.fori_loop(0, _NS // 16, fin_v, 0)
    pltpu.sync_copy(tbuf, v_hbm.at[nsl])
    pltpu.sync_copy(cbuf, vcarr.at[nsl])
    plsc.subcore_barrier()

    pltpu.sync_copy(vcarr, cfull.at[pl.ds(0, _NPAD)])
    cfull[pl.ds(_NPAD, 16)] = jnp.zeros((16,), jnp.float32)
    _gs_pass(vals, dst_flat, src2d, cfull, acc_b, sem)
    plsc.subcore_barrier()

    pltpu.sync_copy(acc_b.at[nsl], sbuf)
    def fin_w(i, _):
        s = pl.ds(i * 16, 16)
        tbuf[s] = abuf[s] * sbuf[s]
        return 0
    lax.fori_loop(0, _NS // 16, fin_w, 0)
    pltpu.sync_copy(tbuf, w_hbm.at[nsl])


_sc_fn = pl.kernel(
    _sc_body,
    out_type=(jax.ShapeDtypeStruct((_NPAD,), jnp.float32),
              jax.ShapeDtypeStruct((_NPAD,), jnp.float32)),
    mesh=plsc.VectorSubcoreMesh(core_axis_name="c", subcore_axis_name="s",
                                num_cores=1, num_subcores=_NSUB),
    compiler_params=pltpu.CompilerParams(needs_layout_passes=False),
    scratch_types=[
        pltpu.VMEM((_ECP,), jnp.int32),
        pltpu.VMEM((_ECP,), jnp.int32),
        pltpu.VMEM((_ROWS, 128), jnp.int32),
        pltpu.VMEM((_ROWS, 128), jnp.int32),
        pltpu.VMEM((_ECP,), jnp.float32),
        pltpu.VMEM((_NPAD + 16,), jnp.float32),
        pltpu.VMEM((_NS,), jnp.float32),
        pltpu.VMEM((_NS,), jnp.float32),
        pltpu.VMEM((_NS,), jnp.float32),
        pltpu.VMEM((_NS,), jnp.float32),
        pltpu.VMEM((_NS,), jnp.float32),
        pltpu.SemaphoreType.DMA,
        pltpu.VMEM_SHARED((_NPAD,), jnp.float32),
        pltpu.VMEM_SHARED((_NPAD,), jnp.float32),
        pltpu.VMEM_SHARED((_NPAD,), jnp.float32),
        pltpu.VMEM_SHARED((_NPAD,), jnp.float32),
    ],
)


def _tc_body(x_ref, w_ref, v_ref, w1_ref, b1_ref, w2_ref, b2_ref,
             wo_ref, bo_ref, o_ref):
    wx = jnp.sum(x_ref[...] * w_ref[...], axis=0, keepdims=True)
    sv = jnp.sum(v_ref[...])
    mm = lambda a, b: lax.dot_general(a, b, (((1,), (0,)), ((), ())),
                                      precision=lax.Precision.HIGHEST)
    t1 = mm(wx, w1_ref[...]) + sv * b1_ref[...]
    t2 = mm(t1, w2_ref[...]) + jnp.float32(_N) * b2_ref[...]
    o_ref[...] = mm(t2, wo_ref[...]) + jnp.float32(_N) * bo_ref[...]


_tc_fn = pl.pallas_call(
    _tc_body,
    out_shape=jax.ShapeDtypeStruct((1, 64), jnp.float32),
)


def kernel(x, edge_index, W1, b1, W2, b2, Wout, bout):
    src = edge_index[0]
    dst = edge_index[1]
    v_pad, w_pad = _sc_fn(src, dst)
    out = _tc_fn(x, w_pad[:_N].reshape(_N, 1), v_pad.reshape(_NPAD // 128, 128),
                 W1, b1.reshape(1, -1), W2, b2.reshape(1, -1),
                 Wout, bout.reshape(1, -1))
    return out[0]


# speedup vs baseline: 1.2589x; 1.0656x over previous
import jax
import jax.numpy as jnp
from jax import lax
from jax.experimental import pallas as pl
from jax.experimental.pallas import tpu as pltpu
from jax.experimental.pallas import tpu_sc as plsc

_N = 10000
_E = 320000
_NPAD = 10240
_NSUB = 16
_EC = _E // _NSUB
_ROWS = 160
_ECP = _ROWS * 128
_FULL = _EC // 128
_REM = _EC - _FULL * 128
_NS = _NPAD // _NSUB


def _rsqrt16(d):
    i = lax.bitcast_convert_type(d, jnp.int32)
    i = jnp.int32(0x5F3759DF) - lax.shift_right_arithmetic(i, 1)
    y = lax.bitcast_convert_type(i, jnp.float32)
    for _ in range(3):
        y = y * (1.5 - 0.5 * d * y * y)
    return y


def _drain_row(vals, idx2d, acc, sem, j):
    pltpu.make_async_copy(vals.at[pl.ds(j * 128, 128)],
                          acc.at[idx2d.at[j]], sem).wait()


def _gs_pass(vals, dst_flat, src2d, cfull, acc, sem):
    G = 8
    def body(g, _):
        j0 = g * G
        def grow(j, _):
            def gchunk(k, _):
                s = pl.ds(j * 128 + k * 16, 16)
                vals[s] = plsc.load_gather(cfull, [dst_flat[s]])
                return 0
            lax.fori_loop(0, 8, gchunk, 0)
            return 0
        lax.fori_loop(j0, j0 + G, grow, 0)
        for t in range(G):
            pltpu.async_copy(vals.at[pl.ds((j0 + t) * 128, 128)],
                             acc.at[src2d.at[j0 + t]], sem, add=True)
        @pl.when(g > 0)
        def _():
            for t in range(G):
                _drain_row(vals, src2d, acc, sem, j0 - G + t)
        return 0
    lax.fori_loop(0, _ROWS // G, body, 0)
    for t in range(G):
        _drain_row(vals, src2d, acc, sem, _ROWS - G + t)


def _sc_body(src_hbm, dst_hbm, v_hbm, w_hbm,
             src_flat, dst_flat, src2d, dst2d, vals, cfull, onebuf,
             zbuf, abuf, cbuf, sbuf, tbuf, sem,
             acc_a, acc_b, carr, vcarr):
    wid = lax.axis_index("s")
    ebase = wid * _EC
    nbase = wid * _NS
    nsl = pl.ds(nbase, _NS)

    def z16(i, _):
        zbuf[pl.ds(i * 16, 16)] = jnp.zeros((16,), jnp.float32)
        return 0
    lax.fori_loop(0, _NS // 16, z16, 0)
    pltpu.sync_copy(zbuf, acc_a.at[nsl])
    pltpu.sync_copy(zbuf, acc_b.at[nsl])

    d_src = pltpu.async_copy(src_hbm.at[pl.ds(ebase, _EC)],
                             src_flat.at[pl.ds(0, _EC)], sem)
    d_dst = pltpu.async_copy(dst_hbm.at[pl.ds(ebase, _EC)],
                             dst_flat.at[pl.ds(0, _EC)], sem)
    d_src.wait()
    d_dst.wait()

    def repack(j, _):
        def inner(k, _):
            s = pl.ds(j * 128 + k * 16, 16)
            d = pl.ds(k * 16, 16)
            src2d[j, d] = src_flat[s]
            dst2d[j, d] = dst_flat[s]
            return 0
        lax.fori_loop(0, 8, inner, 0)
        return 0
    lax.fori_loop(0, _FULL, repack, 0)
    pv = jnp.int32(_N) + (wid * 16 + lax.iota(jnp.int32, 16)) % (_NPAD - _N)
    for k in range(_REM // 16):
        s = pl.ds(_FULL * 128 + k * 16, 16)
        src2d[_FULL, pl.ds(k * 16, 16)] = src_flat[s]
        dst2d[_FULL, pl.ds(k * 16, 16)] = dst_flat[s]
    for j in range(_FULL, _ROWS):
        for k in range((_REM // 16) if j == _FULL else 0, 8):
            src2d[j, pl.ds(k * 16, 16)] = pv
            dst2d[j, pl.ds(k * 16, 16)] = pv
    def padfill(i, _):
        dst_flat[pl.ds(i * 16, 16)] = jnp.full((16,), _NPAD, jnp.int32)
        return 0
    lax.fori_loop(_EC // 16, _ECP // 16, padfill, 0)

    for k in range(8):
        onebuf[pl.ds(k * 16, 16)] = jnp.ones((16,), jnp.float32)
        onebuf[pl.ds(128 + k * 16, 16)] = (
            jnp.ones((16,), jnp.float32) if k < _REM // 16
            else jnp.zeros((16,), jnp.float32))
        onebuf[pl.ds(256 + k * 16, 16)] = jnp.zeros((16,), jnp.float32)

    def hrow(j):
        sel = jnp.where(j < _FULL, 0, jnp.where(j == _FULL, 128, 256))
        return onebuf.at[pl.ds(sel, 128)]

    plsc.subcore_barrier()

    def hist(g, _):
        j0 = g * 4
        for t in range(4):
            pltpu.async_copy(hrow(j0 + t), acc_a.at[src2d.at[j0 + t]],
                             sem, add=True)
            pltpu.async_copy(hrow(j0 + t), acc_b.at[dst2d.at[j0 + t]],
                             sem, add=True)
        @pl.when(g > 0)
        def _():
            for t in range(4):
                j = j0 - 4 + t
                pltpu.make_async_copy(hrow(j), acc_a.at[src2d.at[j]],
                                      sem).wait()
                pltpu.make_async_copy(hrow(j), acc_b.at[dst2d.at[j]],
                                      sem).wait()
        return 0
    lax.fori_loop(0, _ROWS // 4, hist, 0)
    for t in range(4):
        j = _ROWS - 4 + t
        pltpu.make_async_copy(hrow(j), acc_a.at[src2d.at[j]], sem).wait()
        pltpu.make_async_copy(hrow(j), acc_b.at[dst2d.at[j]], sem).wait()
    plsc.subcore_barrier()

    pltpu.sync_copy(acc_a.at[nsl], sbuf)
    def fin_a(i, _):
        s = pl.ds(i * 16, 16)
        abuf[s] = _rsqrt16(jnp.maximum(sbuf[s], 1.0))
        return 0
    lax.fori_loop(0, _NS // 16, fin_a, 0)
    pltpu.sync_copy(acc_b.at[nsl], sbuf)
    def fin_c(i, _):
        s = pl.ds(i * 16, 16)
        cbuf[s] = _rsqrt16(jnp.maximum(sbuf[s], 1.0))
        return 0
    lax.fori_loop(0, _NS // 16, fin_c, 0)
    pltpu.sync_copy(cbuf, carr.at[nsl])
    pltpu.sync_copy(zbuf, acc_a.at[nsl])
    pltpu.sync_copy(zbuf, acc_b.at[nsl])
    plsc.subcore_barrier()

    pltpu.sync_copy(carr, cfull.at[pl.ds(0, _NPAD)])
    cfull[pl.ds(_NPAD, 16)] = jnp.zeros((16,), jnp.float32)
    _gs_pass(vals, dst_flat, src2d, cfull, acc_a, sem)
    plsc.subcore_barrier()

    pltpu.sync_copy(acc_a.at[nsl], sbuf)
    def fin_v(i, _):
        s = pl.ds(i * 16, 16)
        vv = abuf[s] * sbuf[s]
        tbuf[s] = vv
        cbuf[s] = vv * cbuf[s]
        return 0
    lax.fori_loop(0, _NS // 16, fin_v, 0)
    pltpu.sync_copy(tbuf, v_hbm.at[nsl])
    pltpu.sync_copy(cbuf, vcarr.at[nsl])
    plsc.subcore_barrier()

    pltpu.sync_copy(vcarr, cfull.at[pl.ds(0, _NPAD)])
    cfull[pl.ds(_NPAD, 16)] = jnp.zeros((16,), jnp.float32)
    _gs_pass(vals, dst_flat, src2d, cfull, acc_b, sem)
    plsc.subcore_barrier()

    pltpu.sync_copy(acc_b.at[nsl], sbuf)
    def fin_w(i, _):
        s = pl.ds(i * 16, 16)
        tbuf[s] = abuf[s] * sbuf[s]
        return 0
    lax.fori_loop(0, _NS // 16, fin_w, 0)
    pltpu.sync_copy(tbuf, w_hbm.at[nsl])


_sc_fn = pl.kernel(
    _sc_body,
    out_type=(jax.ShapeDtypeStruct((_NPAD,), jnp.float32),
              jax.ShapeDtypeStruct((_NPAD,), jnp.float32)),
    mesh=plsc.VectorSubcoreMesh(core_axis_name="c", subcore_axis_name="s",
                                num_cores=1, num_subcores=_NSUB),
    compiler_params=pltpu.CompilerParams(needs_layout_passes=False),
    scratch_types=[
        pltpu.VMEM((_ECP,), jnp.int32),
        pltpu.VMEM((_ECP,), jnp.int32),
        pltpu.VMEM((_ROWS, 128), jnp.int32),
        pltpu.VMEM((_ROWS, 128), jnp.int32),
        pltpu.VMEM((_ECP,), jnp.float32),
        pltpu.VMEM((_NPAD + 16,), jnp.float32),
        pltpu.VMEM((384,), jnp.float32),
        pltpu.VMEM((_NS,), jnp.float32),
        pltpu.VMEM((_NS,), jnp.float32),
        pltpu.VMEM((_NS,), jnp.float32),
        pltpu.VMEM((_NS,), jnp.float32),
        pltpu.VMEM((_NS,), jnp.float32),
        pltpu.SemaphoreType.DMA,
        pltpu.VMEM_SHARED((_NPAD,), jnp.float32),
        pltpu.VMEM_SHARED((_NPAD,), jnp.float32),
        pltpu.VMEM_SHARED((_NPAD,), jnp.float32),
        pltpu.VMEM_SHARED((_NPAD,), jnp.float32),
    ],
)


def _tc_body(x_ref, w_ref, v_ref, w1_ref, b1_ref, w2_ref, b2_ref,
             wo_ref, bo_ref, o_ref):
    wx = jnp.sum(x_ref[...] * w_ref[...], axis=0, keepdims=True)
    sv = jnp.sum(v_ref[...])
    mm = lambda a, b: lax.dot_general(a, b, (((1,), (0,)), ((), ())),
                                      precision=lax.Precision.HIGHEST)
    t1 = mm(wx, w1_ref[...]) + sv * b1_ref[...]
    t2 = mm(t1, w2_ref[...]) + jnp.float32(_N) * b2_ref[...]
    o_ref[...] = mm(t2, wo_ref[...]) + jnp.float32(_N) * bo_ref[...]


_tc_fn = pl.pallas_call(
    _tc_body,
    out_shape=jax.ShapeDtypeStruct((1, 64), jnp.float32),
)


def kernel(x, edge_index, W1, b1, W2, b2, Wout, bout):
    src = edge_index[0]
    dst = edge_index[1]
    v_pad, w_pad = _sc_fn(src, dst)
    out = _tc_fn(x, w_pad[:_N].reshape(_N, 1), v_pad.reshape(_NPAD // 128, 128),
                 W1, b1.reshape(1, -1), W2, b2.reshape(1, -1),
                 Wout, bout.reshape(1, -1))
    return out[0]
